# baseline (speedup 1.0000x reference)
import functools

import jax
import jax.numpy as jnp
from jax import lax
from jax.experimental import pallas as pl
from jax.experimental.pallas import tpu as pltpu
from jax.experimental.pallas import tpu_sc as plsc

_L = 16
_U = 4


@functools.lru_cache(maxsize=None)
def _make_embed(D, B, S):
    info = plsc.get_sparse_core_info()
    NC, NS = info.num_cores, info.num_subcores
    NW = NC * NS
    POS_PER_W = S // NW
    KP = 8
    NCH = POS_PER_W // KP
    NBUF = B
    NIT = NCH * B
    IPR = D // (_U * _L)

    mesh = plsc.VectorSubcoreMesh(core_axis_name="c", subcore_axis_name="s")

    scratch = (
        [pltpu.VMEM((B * POS_PER_W,), jnp.int32)]
        + [pltpu.VMEM((KP, D), jnp.float32) for _ in range(2)]
        + [pltpu.VMEM((KP, D), jnp.float32) for _ in range(NBUF)]
        + [pltpu.SemaphoreType.DMA for _ in range(2 + 2 * NBUF)]
    )

    @functools.partial(
        pl.kernel,
        mesh=mesh,
        out_type=jax.ShapeDtypeStruct((B, S, D), jnp.float32),
        scratch_types=scratch,
    )
    def body(ids_hbm, wte_hbm, wpe_hbm, out_hbm, idx_v, *bufs):
        pbufs = bufs[0:2]
        gbufs = bufs[2:2 + NBUF]
        psems = bufs[2 + NBUF:4 + NBUF]
        gsems = bufs[4 + NBUF:4 + 2 * NBUF]
        osems = bufs[4 + 2 * NBUF:4 + 3 * NBUF]

        wid = lax.axis_index("s") * NC + lax.axis_index("c")
        p0 = wid * POS_PER_W
        for b in range(B):
            pltpu.sync_copy(ids_hbm.at[b, pl.ds(p0, POS_PER_W)],
                            idx_v.at[pl.ds(b * POS_PER_W, POS_PER_W)])

        def pbuf_copy(pc, par):
            return pltpu.make_async_copy(
                wpe_hbm.at[pl.ds(p0 + pc * KP, KP)], pbufs[par], psems[par])

        def gather_copy(pc, b):
            idx = idx_v.at[pl.ds(b * POS_PER_W + pc * KP, KP)]
            return pltpu.make_async_copy(
                wte_hbm.at[idx], gbufs[b], gsems[b])

        def store_copy(pc, b):
            return pltpu.make_async_copy(
                gbufs[b], out_hbm.at[b, pl.ds(p0 + pc * KP, KP)], osems[b])

        def add_chunk(b, par):
            gb, pb = gbufs[b], pbufs[par]

            @plsc.parallel_loop(0, KP * IPR, 1, unroll=4)
            def addall(jj, gb=gb, pb=pb):
                r = jj // IPR
                c0 = (jj % IPR) * (_U * _L)
                for u in range(_U):
                    sl = pl.ds(c0 + u * _L, _L)
                    plsc.addupdate(gb.at[r, sl], pb[r, sl])

        pbuf_copy(0, 0).start()
        for b in range(NBUF - 1):
            gather_copy(0, b).start()

        def ahead(half, b):
            d, b2 = divmod(B * half + b + NBUF - 1, B)
            return d, b2

        def trip(t, _):
            for half in range(2):
                pc = 2 * t + half
                par = half
                for b in range(B):
                    i = B * (2 * t + half) + b
                    if b == 0:
                        if half == 0:
                            pbuf_copy(pc + 1, 1).start()
                        else:
                            @pl.when(t < NCH // 2 - 1)
                            def _():
                                pbuf_copy(pc + 1, 0).start()
                        pbuf_copy(pc, par).wait()
                    gather_copy(pc, b).wait()
                    add_chunk(b, par)
                    store_copy(pc, b).start()
                    d, b2 = ahead(half, b)
                    if half == 0:
                        @pl.when(i >= 1)
                        def _():
                            pcp, bp = (pc, b - 1) if b else (pc - 1, B - 1)
                            store_copy(pcp, bp).wait()
                        gather_copy(2 * t + d, b2).start()
                    else:
                        @pl.when(i + NBUF - 1 < NIT)
                        def _():
                            pcp, bp = (pc, b - 1) if b else (pc - 1, B - 1)
                            store_copy(pcp, bp).wait()
                            gather_copy(2 * t + d, b2).start()
            return 0

        lax.fori_loop(0, NCH // 2, trip, 0)
        for b in range(B):
            store_copy(NCH - 1, b).wait()

    return body


def kernel(input_ids, attention_mask, wte, wpe):
    B_, S_ = input_ids.shape
    D_ = wte.shape[1]
    fn = _make_embed(D_, B_, S_)
    hidden = fn(input_ids.astype(jnp.int32), wte, wpe)
    return (hidden, input_ids.reshape(-1, S_), attention_mask)


# speedup vs baseline: 1.0323x; 1.0323x over previous
import functools

import jax
import jax.numpy as jnp
from jax import lax
from jax.experimental import pallas as pl
from jax.experimental.pallas import tpu as pltpu
from jax.experimental.pallas import tpu_sc as plsc

_L = 16
_U = 4


@functools.lru_cache(maxsize=None)
def _make_embed(D, B, S):
    info = plsc.get_sparse_core_info()
    NC, NS = info.num_cores, info.num_subcores
    NW = NC * NS
    POS_PER_W = S // NW
    KP = 8
    NCH = POS_PER_W // KP
    NBUF = B
    NIT = NCH * B
    IPR = D // (_U * _L)

    mesh = plsc.VectorSubcoreMesh(core_axis_name="c", subcore_axis_name="s")

    scratch = (
        [pltpu.VMEM((B * POS_PER_W,), jnp.int32)]
        + [pltpu.VMEM((KP, D), jnp.float32) for _ in range(2)]
        + [pltpu.VMEM((KP, D), jnp.float32) for _ in range(NBUF)]
        + [pltpu.SemaphoreType.DMA for _ in range(2 + 2 * NBUF)]
    )

    @functools.partial(
        pl.kernel,
        mesh=mesh,
        out_type=jax.ShapeDtypeStruct((B, S, D), jnp.float32),
        scratch_types=scratch,
    )
    def body(ids_hbm, wte_hbm, wpe_hbm, out_hbm, idx_v, *bufs):
        pbufs = bufs[0:2]
        gbufs = bufs[2:2 + NBUF]
        psems = bufs[2 + NBUF:4 + NBUF]
        gsems = bufs[4 + NBUF:4 + 2 * NBUF]
        osems = bufs[4 + 2 * NBUF:4 + 3 * NBUF]

        wid = lax.axis_index("s") * NC + lax.axis_index("c")
        p0 = wid * POS_PER_W
        ids_cp = [
            pltpu.make_async_copy(ids_hbm.at[b, pl.ds(p0, POS_PER_W)],
                                  idx_v.at[pl.ds(b * POS_PER_W, POS_PER_W)],
                                  bufs[4 + 2 * B + b])
            for b in range(B)
        ]
        for c in ids_cp:
            c.start()

        def pbuf_copy(pc, par):
            return pltpu.make_async_copy(
                wpe_hbm.at[pl.ds(p0 + pc * KP, KP)], pbufs[par], psems[par])

        def gather_copy(pc, b):
            idx = idx_v.at[pl.ds(b * POS_PER_W + pc * KP, KP)]
            return pltpu.make_async_copy(
                wte_hbm.at[idx], gbufs[b], gsems[b])

        def store_copy(pc, b):
            return pltpu.make_async_copy(
                gbufs[b], out_hbm.at[b, pl.ds(p0 + pc * KP, KP)], osems[b])

        def add_chunk(b, par):
            gb, pb = gbufs[b], pbufs[par]

            def addall(jj, _, gb=gb, pb=pb):
                r = jj // IPR
                c0 = (jj % IPR) * (_U * _L)
                for u in range(_U):
                    sl = pl.ds(c0 + u * _L, _L)
                    plsc.addupdate(gb.at[r, sl], pb[r, sl])
                return 0
            lax.fori_loop(0, KP * IPR, addall, 0)

        pbuf_copy(0, 0).start()
        for b in range(NBUF - 1):
            ids_cp[b].wait()
            gather_copy(0, b).start()
        ids_cp[NBUF - 1].wait()

        def ahead(half, b):
            d, b2 = divmod(B * half + b + NBUF - 1, B)
            return d, b2

        def trip(t, _):
            for half in range(2):
                pc = 2 * t + half
                par = half
                for b in range(B):
                    i = B * (2 * t + half) + b
                    if b == 0:
                        if half == 0:
                            pbuf_copy(pc + 1, 1).start()
                        else:
                            @pl.when(t < NCH // 2 - 1)
                            def _():
                                pbuf_copy(pc + 1, 0).start()
                        pbuf_copy(pc, par).wait()
                    gather_copy(pc, b).wait()
                    add_chunk(b, par)
                    store_copy(pc, b).start()
                    d, b2 = ahead(half, b)
                    if half == 0:
                        @pl.when(i >= 1)
                        def _():
                            pcp, bp = (pc, b - 1) if b else (pc - 1, B - 1)
                            store_copy(pcp, bp).wait()
                        gather_copy(2 * t + d, b2).start()
                    else:
                        @pl.when(i + NBUF - 1 < NIT)
                        def _():
                            pcp, bp = (pc, b - 1) if b else (pc - 1, B - 1)
                            store_copy(pcp, bp).wait()
                            gather_copy(2 * t + d, b2).start()
            return 0

        lax.fori_loop(0, NCH // 2, trip, 0)
        for b in range(B):
            store_copy(NCH - 1, b).wait()

    return body


def kernel(input_ids, attention_mask, wte, wpe):
    B_, S_ = input_ids.shape
    D_ = wte.shape[1]
    fn = _make_embed(D_, B_, S_)
    hidden = fn(input_ids.astype(jnp.int32), wte, wpe)
    return (hidden, input_ids.reshape(-1, S_), attention_mask)
